# baseline (speedup 1.0000x reference)
import functools

import jax
import jax.numpy as jnp
from jax import lax
from jax.experimental import pallas as pl
from jax.experimental.pallas import tpu as pltpu
from jax.experimental.pallas import tpu_sc as plsc

_NUM_TYPES = 8
_IN_F = 1024
_OUT_F = 1024
_N = 2048

_BLK = 256
_NB = _N // _BLK + _NUM_TYPES
_TOTAL = _NB * _BLK

_NC = 2
_NS = 16
_NW = _NC * _NS


@functools.lru_cache(maxsize=None)
def _make_sc_dispatch(num_rows, num_slots, feat):
    n_per_w = num_rows // _NW
    assert num_rows % _NW == 0 and n_per_w % 8 == 0
    mesh = plsc.VectorSubcoreMesh(
        core_axis_name="c", subcore_axis_name="s",
        num_cores=_NC, num_subcores=_NS)

    @functools.partial(
        pl.kernel, mesh=mesh,
        out_type=jax.ShapeDtypeStruct((num_slots, feat), jnp.float32),
        scratch_types=[
            pltpu.VMEM((n_per_w,), jnp.int32),
            pltpu.VMEM((n_per_w, feat), jnp.float32),
            pltpu.SemaphoreType.DMA,
        ],
    )
    def dispatch_k(x_hbm, slot_hbm, out_hbm, idx_v, rows_v, sem):
        wid = lax.axis_index("s") * _NC + lax.axis_index("c")
        base = wid * n_per_w
        pltpu.sync_copy(slot_hbm.at[pl.ds(base, n_per_w)], idx_v)
        pltpu.sync_copy(x_hbm.at[pl.ds(base, n_per_w)], rows_v)
        pltpu.async_copy(rows_v, out_hbm.at[idx_v], sem).wait()

    return dispatch_k


@functools.lru_cache(maxsize=None)
def _make_sc_gather(num_idx, feat):
    b_per_w = num_idx // _NW
    assert num_idx % _NW == 0 and b_per_w % 8 == 0
    mesh = plsc.VectorSubcoreMesh(
        core_axis_name="c", subcore_axis_name="s",
        num_cores=_NC, num_subcores=_NS)

    @functools.partial(
        pl.kernel, mesh=mesh,
        out_type=jax.ShapeDtypeStruct((num_idx, feat), jnp.float32),
        scratch_types=[
            pltpu.VMEM((b_per_w,), jnp.int32),
            pltpu.VMEM((b_per_w, feat), jnp.float32),
            pltpu.SemaphoreType.DMA,
        ],
    )
    def gather_k(table_hbm, idx_hbm, out_hbm, idx_v, rows_v, sem):
        wid = lax.axis_index("s") * _NC + lax.axis_index("c")
        base = wid * b_per_w
        pltpu.sync_copy(idx_hbm.at[pl.ds(base, b_per_w)], idx_v)
        pltpu.async_copy(table_hbm.at[idx_v], rows_v, sem).wait()
        pltpu.sync_copy(rows_v, out_hbm.at[pl.ds(base, b_per_w)])

    return gather_k


def _route_body(t_ref, slot_ref, sc_ref):
    t2d = t_ref[...]
    t8 = jnp.concatenate([t2d] * _NUM_TYPES, axis=0)
    ri = lax.broadcasted_iota(jnp.int32, (128, 128), 0)
    ci = lax.broadcasted_iota(jnp.int32, (128, 128), 1)
    rt = ri // 16
    rg = ri % 16
    ct = ci // 16
    cg = ci % 16
    oh = (t8 == rt).astype(jnp.float32)
    upper = (ri <= ci).astype(jnp.float32)
    inner = lax.dot_general(oh, upper, (((1,), (0,)), ((), ())),
                            preferred_element_type=jnp.float32)
    tot_r = inner[:, 127:128]
    tot_c = inner[:, 127].reshape(1, 128)
    e_off = jnp.sum(((rt == ct) & (cg < rg)).astype(jnp.float32) * tot_c,
                    axis=1, keepdims=True)
    rank = inner + e_off
    cnt = jnp.sum((rt == ct).astype(jnp.float32) * tot_c,
                  axis=1, keepdims=True)
    padded = (((cnt + 0.5).astype(jnp.int32) + _BLK - 1) // _BLK) * _BLK
    paddedf = padded.astype(jnp.float32)
    pstart = jnp.sum(((ct < rt) & (cg == 0)).astype(jnp.float32) *
                     paddedf.reshape(1, 128), axis=1, keepdims=True)
    contrib = (pstart - 1.0 + rank) * oh
    gi16 = lax.broadcasted_iota(jnp.int32, (16, 128), 0)
    rj16 = lax.broadcasted_iota(jnp.int32, (16, 128), 1)
    gsel = ((rj16 % 16) == gi16).astype(jnp.float32)
    slot2d = lax.dot_general(gsel, contrib, (((1,), (0,)), ((), ())),
                             preferred_element_type=jnp.float32)
    slot_ref[...] = (slot2d + 0.5).astype(jnp.int32)

    r1 = lax.broadcasted_iota(jnp.int32, (128, 1), 0)
    g1 = r1 % 16
    t1 = r1 // 16
    mask0 = (g1 == 0).astype(jnp.float32)
    totpad = jnp.sum(paddedf * mask0)
    nact_i = totpad.astype(jnp.int32) // _BLK
    bidx = lax.broadcasted_iota(jnp.int32, (1, _NB), 1)
    blkf = (bidx * _BLK).astype(jnp.float32)
    maskp = ((g1 == 0) & (t1 >= 1)).astype(jnp.float32)
    bt = jnp.sum((pstart <= blkf).astype(jnp.float32) * maskp,
                 axis=0, keepdims=True).astype(jnp.int32)
    lastref = jnp.maximum(nact_i - 1, 0)
    last = jnp.sum(bt * (bidx == lastref).astype(jnp.int32))
    active = (bidx * _BLK) < totpad.astype(jnp.int32)
    bt = jnp.where(active, bt, last)
    xi = jnp.where(active, bidx, lastref)
    oi = jnp.where(active, bidx, _NB - 1)
    nv = jnp.full((1, 16), nact_i, jnp.int32)
    sc_ref[...] = jnp.concatenate([bt, xi, oi, nv], axis=1)


def _mm_body(s_ref, x_ref, w_ref, b_ref, o_ref):
    i = pl.program_id(0)

    @pl.when(i < s_ref[3 * _NB])
    def _():
        acc = lax.dot_general(
            x_ref[...], w_ref[0],
            (((1,), (1,)), ((), ())),
            preferred_element_type=jnp.float32,
        )
        o_ref[...] = acc + b_ref[0]


_grid_spec = pltpu.PrefetchScalarGridSpec(
    num_scalar_prefetch=1,
    grid=(_NB,),
    in_specs=[
        pl.BlockSpec((_BLK, _IN_F), lambda i, s: (s[_NB + i], 0)),
        pl.BlockSpec((1, _OUT_F, _IN_F), lambda i, s: (s[i], 0, 0)),
        pl.BlockSpec((1, 1, _OUT_F), lambda i, s: (s[i], 0, 0)),
    ],
    out_specs=pl.BlockSpec((_BLK, _OUT_F), lambda i, s: (s[2 * _NB + i], 0)),
)


def kernel(x, types, W, b):
    slot2d, sc = pl.pallas_call(
        _route_body,
        out_shape=[
            jax.ShapeDtypeStruct((16, 128), jnp.int32),
            jax.ShapeDtypeStruct((1, 64), jnp.int32),
        ],
    )(types.astype(jnp.int32).reshape(16, 128))
    slot = slot2d.reshape(_N)
    scalars = sc.reshape(64)

    x_pad = _make_sc_dispatch(_N, _TOTAL, _IN_F)(x, slot)
    out_pad = pl.pallas_call(
        _mm_body,
        grid_spec=_grid_spec,
        out_shape=jax.ShapeDtypeStruct((_TOTAL, _OUT_F), jnp.float32),
        compiler_params=pltpu.CompilerParams(
            dimension_semantics=("parallel",)),
    )(scalars, x_pad, W, b.reshape(_NUM_TYPES, 1, _OUT_F))
    return _make_sc_gather(_N, _OUT_F)(out_pad, slot)


# speedup vs baseline: 1.0608x; 1.0608x over previous
import functools

import jax
import jax.numpy as jnp
from jax import lax
from jax.experimental import pallas as pl
from jax.experimental.pallas import tpu as pltpu
from jax.experimental.pallas import tpu_sc as plsc

_NUM_TYPES = 8
_IN_F = 1024
_OUT_F = 1024
_N = 2048

_BLK = 256
_NB = _N // _BLK + _NUM_TYPES
_TOTAL = _NB * _BLK

_NC = 2
_NS = 16
_NW = _NC * _NS


@functools.lru_cache(maxsize=None)
def _make_sc_dispatch(num_rows, num_slots, feat):
    n_per_w = num_rows // _NW
    assert num_rows % _NW == 0 and n_per_w % 8 == 0
    mesh = plsc.VectorSubcoreMesh(
        core_axis_name="c", subcore_axis_name="s",
        num_cores=_NC, num_subcores=_NS)

    @functools.partial(
        pl.kernel, mesh=mesh,
        out_type=jax.ShapeDtypeStruct((num_slots, feat), jnp.float32),
        scratch_types=[
            pltpu.VMEM((n_per_w,), jnp.int32),
            pltpu.VMEM((n_per_w, feat), jnp.float32),
            pltpu.SemaphoreType.DMA,
        ],
    )
    def dispatch_k(x_hbm, slot_hbm, out_hbm, idx_v, rows_v, sem):
        wid = lax.axis_index("s") * _NC + lax.axis_index("c")
        base = wid * n_per_w
        pltpu.sync_copy(slot_hbm.at[pl.ds(base, n_per_w)], idx_v)
        pltpu.sync_copy(x_hbm.at[pl.ds(base, n_per_w)], rows_v)
        pltpu.async_copy(rows_v, out_hbm.at[idx_v], sem).wait()

    return dispatch_k


@functools.lru_cache(maxsize=None)
def _make_sc_gather(num_idx, feat):
    b_per_w = num_idx // _NW
    assert num_idx % _NW == 0 and b_per_w % 8 == 0
    mesh = plsc.VectorSubcoreMesh(
        core_axis_name="c", subcore_axis_name="s",
        num_cores=_NC, num_subcores=_NS)

    @functools.partial(
        pl.kernel, mesh=mesh,
        out_type=jax.ShapeDtypeStruct((num_idx, feat), jnp.float32),
        scratch_types=[
            pltpu.VMEM((b_per_w,), jnp.int32),
            pltpu.VMEM((b_per_w, feat), jnp.float32),
            pltpu.SemaphoreType.DMA,
        ],
    )
    def gather_k(table_hbm, idx_hbm, out_hbm, idx_v, rows_v, sem):
        wid = lax.axis_index("s") * _NC + lax.axis_index("c")
        base = wid * b_per_w
        pltpu.sync_copy(idx_hbm.at[pl.ds(base, b_per_w)], idx_v)
        pltpu.async_copy(table_hbm.at[idx_v], rows_v, sem).wait()
        pltpu.sync_copy(rows_v, out_hbm.at[pl.ds(base, b_per_w)])

    return gather_k


def _route_body(t_ref, slot_ref, sc_ref):
    t2d = t_ref[...]
    t8 = jnp.concatenate([t2d] * _NUM_TYPES, axis=0)
    ri = lax.broadcasted_iota(jnp.int32, (128, 128), 0)
    ci = lax.broadcasted_iota(jnp.int32, (128, 128), 1)
    rt = ri // 16
    rg = ri % 16
    ct = ci // 16
    cg = ci % 16
    oh = (t8 == rt).astype(jnp.float32)
    upper = (ri <= ci).astype(jnp.float32)
    inner = lax.dot_general(oh, upper, (((1,), (0,)), ((), ())),
                            preferred_element_type=jnp.float32)
    tot_r = inner[:, 127:128]
    tot_c = inner[:, 127].reshape(1, 128)
    e_off = jnp.sum(((rt == ct) & (cg < rg)).astype(jnp.float32) * tot_c,
                    axis=1, keepdims=True)
    rank = inner + e_off
    cnt = jnp.sum((rt == ct).astype(jnp.float32) * tot_c,
                  axis=1, keepdims=True)
    padded = (((cnt + 0.5).astype(jnp.int32) + _BLK - 1) // _BLK) * _BLK
    paddedf = padded.astype(jnp.float32)
    pstart = jnp.sum(((ct < rt) & (cg == 0)).astype(jnp.float32) *
                     paddedf.reshape(1, 128), axis=1, keepdims=True)
    contrib = (pstart - 1.0 + rank) * oh
    slot2d = jnp.sum(contrib.reshape(_NUM_TYPES, 16, 128), axis=0)
    slot_ref[...] = (slot2d + 0.5).astype(jnp.int32)

    r1 = lax.broadcasted_iota(jnp.int32, (128, 1), 0)
    g1 = r1 % 16
    t1 = r1 // 16
    mask0 = (g1 == 0).astype(jnp.float32)
    totpad = jnp.sum(paddedf * mask0)
    nact_i = totpad.astype(jnp.int32) // _BLK
    bidx = lax.broadcasted_iota(jnp.int32, (1, _NB), 1)
    blkf = (bidx * _BLK).astype(jnp.float32)
    maskp = ((g1 == 0) & (t1 >= 1)).astype(jnp.float32)
    bt = jnp.sum((pstart <= blkf).astype(jnp.float32) * maskp,
                 axis=0, keepdims=True).astype(jnp.int32)
    lastref = jnp.maximum(nact_i - 1, 0)
    last = jnp.sum(bt * (bidx == lastref).astype(jnp.int32))
    active = (bidx * _BLK) < totpad.astype(jnp.int32)
    bt = jnp.where(active, bt, last)
    xi = jnp.where(active, bidx, lastref)
    oi = jnp.where(active, bidx, _NB - 1)
    nv = jnp.full((1, 16), nact_i, jnp.int32)
    sc_ref[...] = jnp.concatenate([bt, xi, oi, nv], axis=1)


def _mm_body(s_ref, x_ref, w_ref, b_ref, o_ref):
    i = pl.program_id(0)

    @pl.when(i < s_ref[3 * _NB])
    def _():
        acc = lax.dot_general(
            x_ref[...], w_ref[0],
            (((1,), (1,)), ((), ())),
            preferred_element_type=jnp.float32,
        )
        o_ref[...] = acc + b_ref[0]


_grid_spec = pltpu.PrefetchScalarGridSpec(
    num_scalar_prefetch=1,
    grid=(_NB,),
    in_specs=[
        pl.BlockSpec((_BLK, _IN_F), lambda i, s: (s[_NB + i], 0)),
        pl.BlockSpec((1, _OUT_F, _IN_F), lambda i, s: (s[i], 0, 0)),
        pl.BlockSpec((1, 1, _OUT_F), lambda i, s: (s[i], 0, 0)),
    ],
    out_specs=pl.BlockSpec((_BLK, _OUT_F), lambda i, s: (s[2 * _NB + i], 0)),
)


def kernel(x, types, W, b):
    slot2d, sc = pl.pallas_call(
        _route_body,
        out_shape=[
            jax.ShapeDtypeStruct((16, 128), jnp.int32),
            jax.ShapeDtypeStruct((1, 64), jnp.int32),
        ],
    )(types.astype(jnp.int32).reshape(16, 128))
    slot = slot2d.reshape(_N)
    scalars = sc.reshape(64)

    x_pad = _make_sc_dispatch(_N, _TOTAL, _IN_F)(x, slot)
    out_pad = pl.pallas_call(
        _mm_body,
        grid_spec=_grid_spec,
        out_shape=jax.ShapeDtypeStruct((_TOTAL, _OUT_F), jnp.float32),
        compiler_params=pltpu.CompilerParams(
            dimension_semantics=("parallel",)),
    )(scalars, x_pad, W, b.reshape(_NUM_TYPES, 1, _OUT_F))
    return _make_sc_gather(_N, _OUT_F)(out_pad, slot)


# speedup vs baseline: 1.0648x; 1.0038x over previous
import functools

import jax
import jax.numpy as jnp
from jax import lax
from jax.experimental import pallas as pl
from jax.experimental.pallas import tpu as pltpu
from jax.experimental.pallas import tpu_sc as plsc

_NUM_TYPES = 8
_IN_F = 1024
_OUT_F = 1024
_N = 2048

_BLK = 256
_NB = _N // _BLK + _NUM_TYPES
_TOTAL = _NB * _BLK

_NC = 2
_NS = 16
_NW = _NC * _NS


@functools.lru_cache(maxsize=None)
def _make_sc_dispatch(num_rows, num_slots, feat):
    n_per_w = num_rows // _NW
    assert num_rows % _NW == 0 and n_per_w % 8 == 0
    mesh = plsc.VectorSubcoreMesh(
        core_axis_name="c", subcore_axis_name="s",
        num_cores=_NC, num_subcores=_NS)

    @functools.partial(
        pl.kernel, mesh=mesh,
        out_type=jax.ShapeDtypeStruct((num_slots, feat), jnp.float32),
        scratch_types=[
            pltpu.VMEM((n_per_w,), jnp.int32),
            pltpu.VMEM((n_per_w, feat), jnp.float32),
            pltpu.SemaphoreType.DMA,
        ],
    )
    def dispatch_k(x_hbm, slot_hbm, out_hbm, idx_v, rows_v, sem):
        wid = lax.axis_index("s") * _NC + lax.axis_index("c")
        base = wid * n_per_w
        pltpu.sync_copy(slot_hbm.at[pl.ds(base, n_per_w)], idx_v)
        pltpu.sync_copy(x_hbm.at[pl.ds(base, n_per_w)], rows_v)
        pltpu.async_copy(rows_v, out_hbm.at[idx_v], sem).wait()

    return dispatch_k


@functools.lru_cache(maxsize=None)
def _make_sc_gather(num_idx, feat):
    b_per_w = num_idx // _NW
    assert num_idx % _NW == 0 and b_per_w % 8 == 0
    mesh = plsc.VectorSubcoreMesh(
        core_axis_name="c", subcore_axis_name="s",
        num_cores=_NC, num_subcores=_NS)

    @functools.partial(
        pl.kernel, mesh=mesh,
        out_type=jax.ShapeDtypeStruct((num_idx, feat), jnp.float32),
        scratch_types=[
            pltpu.VMEM((b_per_w,), jnp.int32),
            pltpu.VMEM((b_per_w, feat), jnp.float32),
            pltpu.SemaphoreType.DMA,
        ],
    )
    def gather_k(table_hbm, idx_hbm, out_hbm, idx_v, rows_v, sem):
        wid = lax.axis_index("s") * _NC + lax.axis_index("c")
        base = wid * b_per_w
        pltpu.sync_copy(idx_hbm.at[pl.ds(base, b_per_w)], idx_v)
        pltpu.async_copy(table_hbm.at[idx_v], rows_v, sem).wait()
        pltpu.sync_copy(rows_v, out_hbm.at[pl.ds(base, b_per_w)])

    return gather_k


def _route_body(t_ref, slot_ref, sc_ref):
    t2d = t_ref[...]
    t8 = jnp.concatenate([t2d] * _NUM_TYPES, axis=0)
    ri = lax.broadcasted_iota(jnp.int32, (128, 128), 0)
    ci = lax.broadcasted_iota(jnp.int32, (128, 128), 1)
    rt = ri // 16
    rg = ri % 16
    ct = ci // 16
    cg = ci % 16
    oh = (t8 == rt).astype(jnp.float32)
    upper = (ri <= ci).astype(jnp.float32)
    inner = lax.dot_general(oh, upper, (((1,), (0,)), ((), ())),
                            preferred_element_type=jnp.float32)
    tot_c = inner[:, 127].reshape(1, 128)
    e_off = jnp.sum(((rt == ct) & (cg < rg)).astype(jnp.float32) * tot_c,
                    axis=1, keepdims=True)
    rank = inner + e_off
    cnt = jnp.sum((rt == ct).astype(jnp.float32) * tot_c,
                  axis=1, keepdims=True)
    padded = (((cnt + 0.5).astype(jnp.int32) + _BLK - 1) // _BLK) * _BLK
    paddedf = padded.astype(jnp.float32)
    pstart = jnp.sum(((ct < rt) & (cg == 0)).astype(jnp.float32) *
                     paddedf.reshape(1, 128), axis=1, keepdims=True)
    contrib = (pstart - 1.0 + rank) * oh
    slot2d = jnp.sum(contrib.reshape(_NUM_TYPES, 16, 128), axis=0)
    slot_ref[...] = (slot2d + 0.5).astype(jnp.int32)

    r1 = lax.broadcasted_iota(jnp.int32, (128, 1), 0)
    g1 = r1 % 16
    t1 = r1 // 16
    mask0 = (g1 == 0).astype(jnp.float32)
    totpad = jnp.sum(paddedf * mask0)
    nact_i = totpad.astype(jnp.int32) // _BLK
    bidx = lax.broadcasted_iota(jnp.int32, (1, _NB), 1)
    blkf = (bidx * _BLK).astype(jnp.float32)
    maskp = ((g1 == 0) & (t1 >= 1)).astype(jnp.float32)
    bt = jnp.sum((pstart <= blkf).astype(jnp.float32) * maskp,
                 axis=0, keepdims=True).astype(jnp.int32)
    lastref = jnp.maximum(nact_i - 1, 0)
    last = jnp.sum(bt * (bidx == lastref).astype(jnp.int32))
    active = (bidx * _BLK) < totpad.astype(jnp.int32)
    bt = jnp.where(active, bt, last)
    xi = jnp.where(active, bidx, lastref)
    oi = jnp.where(active, bidx, _NB - 1)
    nv = jnp.full((1, 16), nact_i, jnp.int32)
    sc_ref[...] = jnp.concatenate([bt, xi, oi, nv], axis=1)


def _mm_body(s_ref, x_ref, w_ref, b_ref, o_ref):
    i = pl.program_id(0)

    @pl.when(i < s_ref[3 * _NB])
    def _():
        acc = lax.dot_general(
            x_ref[...], w_ref[0],
            (((1,), (1,)), ((), ())),
            preferred_element_type=jnp.float32,
        )
        o_ref[...] = acc + b_ref[0]


_grid_spec = pltpu.PrefetchScalarGridSpec(
    num_scalar_prefetch=1,
    grid=(_NB,),
    in_specs=[
        pl.BlockSpec((_BLK, _IN_F), lambda i, s: (s[_NB + i], 0)),
        pl.BlockSpec((1, _OUT_F, _IN_F), lambda i, s: (s[i], 0, 0)),
        pl.BlockSpec((1, 1, _OUT_F), lambda i, s: (s[i], 0, 0)),
    ],
    out_specs=pl.BlockSpec((_BLK, _OUT_F), lambda i, s: (s[2 * _NB + i], 0)),
)


def kernel(x, types, W, b):
    slot2d, sc = pl.pallas_call(
        _route_body,
        out_shape=[
            jax.ShapeDtypeStruct((16, 128), jnp.int32),
            jax.ShapeDtypeStruct((1, 64), jnp.int32),
        ],
    )(types.astype(jnp.int32).reshape(16, 128))
    slot = slot2d.reshape(_N)
    scalars = sc.reshape(64)

    x_pad = _make_sc_dispatch(_N, _TOTAL, _IN_F)(x, slot)
    out_pad = pl.pallas_call(
        _mm_body,
        grid_spec=_grid_spec,
        out_shape=jax.ShapeDtypeStruct((_TOTAL, _OUT_F), jnp.float32),
        compiler_params=pltpu.CompilerParams(
            dimension_semantics=("parallel",)),
    )(scalars, x_pad, W, b.reshape(_NUM_TYPES, 1, _OUT_F))
    return _make_sc_gather(_N, _OUT_F)(out_pad, slot)
